# baseline (speedup 1.0000x reference)
import functools

import jax
import jax.numpy as jnp
from jax import lax
from jax.experimental import pallas as pl
from jax.experimental.pallas import tpu as pltpu
from jax.experimental.pallas import tpu_sc as plsc

try:
    _info = plsc.get_sparse_core_info()
    NC, NS = int(_info.num_cores), int(_info.num_subcores)
except Exception:
    NC, NS = 2, 16
NW = NC * NS

_K = 128


def _mila(x):
    return x * jnp.tanh(jax.nn.softplus(x - 1.0))


@functools.lru_cache(maxsize=None)
def _make_sc_gather2(N, C, E, dtype=jnp.float32):
    n_chunks = E // _K
    mesh = plsc.VectorSubcoreMesh(
        core_axis_name="c", subcore_axis_name="s", num_cores=NC, num_subcores=NS
    )
    out = jax.ShapeDtypeStruct((E, C), dtype)

    @functools.partial(
        pl.kernel,
        out_type=(out, out),
        mesh=mesh,
        compiler_params=pltpu.CompilerParams(use_tc_tiling_on_sc=False),
        scratch_types=[
            pltpu.VMEM((_K,), jnp.int32),
            pltpu.VMEM((_K,), jnp.int32),
            pltpu.VMEM((_K, C), dtype),
            pltpu.VMEM((_K, C), dtype),
            pltpu.SemaphoreType.DMA,
            pltpu.SemaphoreType.DMA,
        ],
    )
    def k(table, idx_s, idx_r, out_s, out_r, iv_s, iv_r, rv_s, rv_r, sem_s, sem_r):
        wid = lax.axis_index("s") * NC + lax.axis_index("c")

        @pl.loop(wid, n_chunks, step=NW)
        def _(ci):
            base = ci * _K
            pltpu.sync_copy(idx_s.at[pl.ds(base, _K)], iv_s)
            pltpu.sync_copy(idx_r.at[pl.ds(base, _K)], iv_r)
            cp_s = pltpu.async_copy(table.at[iv_s], rv_s, sem_s)
            cp_r = pltpu.async_copy(table.at[iv_r], rv_r, sem_r)
            cp_s.wait()
            pltpu.sync_copy(rv_s, out_s.at[pl.ds(base, _K)])
            cp_r.wait()
            pltpu.sync_copy(rv_r, out_r.at[pl.ds(base, _K)])

    return k


@functools.lru_cache(maxsize=None)
def _make_sc_scatter(N, C, E, CW):
    n_passes = C // (2 * CW)
    n_chunks = E // _K
    rpt = N // NS
    mesh = plsc.VectorSubcoreMesh(
        core_axis_name="c", subcore_axis_name="s", num_cores=NC, num_subcores=NS
    )

    @functools.partial(
        pl.kernel,
        out_type=jax.ShapeDtypeStruct((N, C), jnp.float32),
        mesh=mesh,
        compiler_params=pltpu.CompilerParams(use_tc_tiling_on_sc=False),
        scratch_types=[
            pltpu.VMEM((_K,), jnp.int32),
            pltpu.VMEM((_K, CW), jnp.float32),
            pltpu.VMEM_SHARED((N, CW), jnp.float32),
        ],
    )
    def k(vals, idx, zeros, out, iv, rv, acc):
        cid = lax.axis_index("c")
        sid = lax.axis_index("s")
        row0 = sid * rpt
        for p in range(n_passes):
            col0 = (2 * p + cid) * CW
            pltpu.sync_copy(zeros.at[pl.ds(row0, rpt)], acc.at[pl.ds(row0, rpt)])
            plsc.subcore_barrier()

            @pl.loop(sid, n_chunks, step=NS)
            def _(ci):
                base = ci * _K
                pltpu.sync_copy(idx.at[pl.ds(base, _K)], iv)
                pltpu.sync_copy(vals.at[pl.ds(base, _K), pl.ds(col0, CW)], rv)
                pltpu.sync_copy(rv, acc.at[iv], add=True)

            plsc.subcore_barrier()
            pltpu.sync_copy(
                acc.at[pl.ds(row0, rpt)],
                out.at[pl.ds(row0, rpt), pl.ds(col0, CW)],
            )
            plsc.subcore_barrier()

    return k


def _embed_body(x_ref, w_ref, b_ref, o_ref, obf_ref=None):
    o = _mila(
        jnp.dot(x_ref[...], w_ref[...], preferred_element_type=jnp.float32)
        + b_ref[...]
    )
    o_ref[...] = o
    if obf_ref is not None:
        obf_ref[...] = o.astype(jnp.bfloat16)


def _tc_embed(x, w, b, blk, with_bf16=False):
    R, Din = x.shape
    Dout = w.shape[1]
    out_specs = pl.BlockSpec((blk, Dout), lambda i: (i, 0))
    out_shape = jax.ShapeDtypeStruct((R, Dout), jnp.float32)
    if with_bf16:
        out_specs = [out_specs, pl.BlockSpec((blk, Dout), lambda i: (i, 0))]
        out_shape = [out_shape, jax.ShapeDtypeStruct((R, Dout), jnp.bfloat16)]
    return pl.pallas_call(
        _embed_body,
        grid=(R // blk,),
        in_specs=[
            pl.BlockSpec((blk, Din), lambda i: (i, 0)),
            pl.BlockSpec((Din, Dout), lambda i: (0, 0)),
            pl.BlockSpec((1, Dout), lambda i: (0, 0)),
        ],
        out_specs=out_specs,
        out_shape=out_shape,
    )(x, w, b)


def _edge_step_body(e_ref, gr_ref, gs_ref, w1_ref, b1_ref, w2_ref, b2_ref, o_ref):
    for br in range(3):
        c0 = br * 64
        x = jnp.concatenate(
            [
                e_ref[:, c0:c0 + 64],
                gr_ref[:, c0:c0 + 64].astype(jnp.float32),
                gs_ref[:, c0:c0 + 64].astype(jnp.float32),
            ],
            axis=1,
        )
        h = _mila(
            jnp.dot(x, w1_ref[br], preferred_element_type=jnp.float32)
            + b1_ref[br]
        )
        o_ref[:, c0:c0 + 64] = _mila(
            jnp.dot(h, w2_ref[br], preferred_element_type=jnp.float32)
            + b2_ref[br]
        )


def _tc_edge_step(e3, gr, gs, w1, b1, w2, b2, blk):
    E = e3.shape[0]
    return pl.pallas_call(
        _edge_step_body,
        grid=(E // blk,),
        in_specs=[
            pl.BlockSpec((blk, 192), lambda i: (i, 0)),
            pl.BlockSpec((blk, 192), lambda i: (i, 0)),
            pl.BlockSpec((blk, 192), lambda i: (i, 0)),
            pl.BlockSpec((3, 192, 64), lambda i: (0, 0, 0)),
            pl.BlockSpec((3, 1, 64), lambda i: (0, 0, 0)),
            pl.BlockSpec((3, 64, 64), lambda i: (0, 0, 0)),
            pl.BlockSpec((3, 1, 64), lambda i: (0, 0, 0)),
        ],
        out_specs=pl.BlockSpec((blk, 192), lambda i: (i, 0)),
        out_shape=jax.ShapeDtypeStruct((E, 192), jnp.float32),
    )(e3, gr, gs, w1, b1, w2, b2)


def _node_step_body(agg_ref, n_ref, w1_ref, b1_ref, w2_ref, b2_ref, o_ref,
                    obf_ref=None):
    for br in range(3):
        c0 = br * 64
        n_b = n_ref[:, c0:c0 + 64]
        x = jnp.concatenate([agg_ref[:, c0:c0 + 64], n_b], axis=1)
        h = _mila(
            jnp.dot(x, w1_ref[br], preferred_element_type=jnp.float32)
            + b1_ref[br]
        )
        o = (
            _mila(
                jnp.dot(h, w2_ref[br], preferred_element_type=jnp.float32)
                + b2_ref[br]
            )
            + n_b
        )
        o_ref[:, c0:c0 + 64] = o
        if obf_ref is not None:
            obf_ref[:, c0:c0 + 64] = o.astype(jnp.bfloat16)


def _tc_node_step(agg, n3, w1, b1, w2, b2, blk, with_bf16=False):
    N = n3.shape[0]
    out_specs = pl.BlockSpec((blk, 192), lambda i: (i, 0))
    out_shape = jax.ShapeDtypeStruct((N, 192), jnp.float32)
    if with_bf16:
        out_specs = [out_specs, pl.BlockSpec((blk, 192), lambda i: (i, 0))]
        out_shape = [out_shape, jax.ShapeDtypeStruct((N, 192), jnp.bfloat16)]
    return pl.pallas_call(
        _node_step_body,
        grid=(N // blk,),
        in_specs=[
            pl.BlockSpec((blk, 192), lambda i: (i, 0)),
            pl.BlockSpec((blk, 192), lambda i: (i, 0)),
            pl.BlockSpec((3, 128, 64), lambda i: (0, 0, 0)),
            pl.BlockSpec((3, 1, 64), lambda i: (0, 0, 0)),
            pl.BlockSpec((3, 64, 64), lambda i: (0, 0, 0)),
            pl.BlockSpec((3, 1, 64), lambda i: (0, 0, 0)),
        ],
        out_specs=out_specs,
        out_shape=out_shape,
    )(agg, n3, w1, b1, w2, b2)


def _mono_head_body(n_ref, w1_ref, b1_ref, w2_ref, b2_ref, w3_ref, b3_ref,
                    raw_ref, acc_ref):
    h = _mila(
        jnp.dot(n_ref[...], w1_ref[...], preferred_element_type=jnp.float32)
        + b1_ref[...]
    )
    h = _mila(
        jnp.dot(h, w2_ref[...], preferred_element_type=jnp.float32)
        + b2_ref[...]
    )
    raw = jnp.dot(h, w3_ref[...], preferred_element_type=jnp.float32) + b3_ref[...]
    raw_ref[...] = raw

    @pl.when(pl.program_id(0) == 0)
    def _():
        acc_ref[...] = jnp.zeros_like(acc_ref)

    acc_ref[...] += jnp.broadcast_to(jnp.sum(raw), acc_ref.shape)


def _tc_mono_head(n_mono, w1, b1, w2, b2, w3, b3, blk):
    N = n_mono.shape[0]
    return pl.pallas_call(
        _mono_head_body,
        grid=(N // blk,),
        in_specs=[
            pl.BlockSpec((blk, 64), lambda i: (i, 0)),
            pl.BlockSpec((64, 64), lambda i: (0, 0)),
            pl.BlockSpec((1, 64), lambda i: (0, 0)),
            pl.BlockSpec((64, 64), lambda i: (0, 0)),
            pl.BlockSpec((1, 64), lambda i: (0, 0)),
            pl.BlockSpec((64, 1), lambda i: (0, 0)),
            pl.BlockSpec((1, 1), lambda i: (0, 0)),
        ],
        out_specs=[
            pl.BlockSpec((blk, 1), lambda i: (i, 0)),
            pl.BlockSpec((8, 128), lambda i: (0, 0)),
        ],
        out_shape=[
            jax.ShapeDtypeStruct((N, 1), jnp.float32),
            jax.ShapeDtypeStruct((8, 128), jnp.float32),
        ],
    )(n_mono, w1, b1, w2, b2, w3, b3)


def _mean_sub_body(raw_ref, acc_ref, o_ref, *, count):
    total = jnp.sum(acc_ref[...]) / (8.0 * 128.0)
    o_ref[...] = raw_ref[...] - total / count


def _tc_mean_sub(raw, acc, blk):
    N = raw.shape[0]
    return pl.pallas_call(
        functools.partial(_mean_sub_body, count=float(N)),
        grid=(N // blk,),
        in_specs=[
            pl.BlockSpec((blk, 1), lambda i: (i, 0)),
            pl.BlockSpec((8, 128), lambda i: (0, 0)),
        ],
        out_specs=pl.BlockSpec((blk, 1), lambda i: (i, 0)),
        out_shape=jax.ShapeDtypeStruct((N, 1), jnp.float32),
    )(raw, acc)


def _edge_head_body(hs_ref, hr_ref, ed_ref,
                    w1d_ref, b1d_ref, w2d_ref, b2d_ref, w3d_ref, b3d_ref,
                    w1q_ref, b1q_ref, w2q_ref, b2q_ref, w3q_ref, b3q_ref,
                    o_ref):
    ed = ed_ref[...]

    def head(off, w1, b1, w2, b2, w3, b3):
        x = jnp.concatenate(
            [hs_ref[:, off:off + 64], hr_ref[:, off:off + 64], ed], axis=1
        )
        h = _mila(jnp.dot(x, w1[...], preferred_element_type=jnp.float32) + b1[...])
        h = _mila(jnp.dot(h, w2[...], preferred_element_type=jnp.float32) + b2[...])
        return jnp.dot(h, w3[...], preferred_element_type=jnp.float32) + b3[...]

    wd = head(0, w1d_ref, b1d_ref, w2d_ref, b2d_ref, w3d_ref, b3d_ref)
    wq = head(64, w1q_ref, b1q_ref, w2q_ref, b2q_ref, w3q_ref, b3q_ref)

    v = hs_ref[:, 128:131] - hr_ref[:, 128:131]
    vx, vy, vz = v[:, 0:1], v[:, 1:2], v[:, 2:3]
    tr3 = (vx * vx + vy * vy + vz * vz) * (1.0 / 3.0)
    q00 = wq * (vx * vx - tr3)
    q11 = wq * (vy * vy - tr3)
    q22 = wq * (vz * vz - tr3)
    q01 = wq * (vx * vy)
    q02 = wq * (vx * vz)
    q12 = wq * (vy * vz)
    zero = jnp.zeros_like(wd)
    o_ref[...] = jnp.concatenate(
        [wd * vx, wd * vy, wd * vz,
         q00, q01, q02, q01, q11, q12, q02, q12, q22,
         zero, zero, zero, zero],
        axis=1,
    )


def _tc_edge_head(hs, hr, edges, pd, pq, blk):
    E = hs.shape[0]
    w1d, b1d, w2d, b2d, w3d, b3d = pd
    w1q, b1q, w2q, b2q, w3q, b3q = pq
    wspec = lambda shp: pl.BlockSpec(shp, lambda i: (0, 0))
    return pl.pallas_call(
        _edge_head_body,
        grid=(E // blk,),
        in_specs=[
            pl.BlockSpec((blk, 144), lambda i: (i, 0)),
            pl.BlockSpec((blk, 144), lambda i: (i, 0)),
            pl.BlockSpec((blk, 32), lambda i: (i, 0)),
            wspec((160, 64)), wspec((1, 64)), wspec((64, 64)), wspec((1, 64)),
            wspec((64, 1)), wspec((1, 1)),
            wspec((160, 64)), wspec((1, 64)), wspec((64, 64)), wspec((1, 64)),
            wspec((64, 1)), wspec((1, 1)),
        ],
        out_specs=pl.BlockSpec((blk, 16), lambda i: (i, 0)),
        out_shape=jax.ShapeDtypeStruct((E, 16), jnp.float32),
    )(hs, hr, edges, w1d, b1d, w2d, b2d, w3d, b3d,
      w1q, b1q, w2q, b2q, w3q, b3q)


_BRANCHES = ("mono", "dipo", "quad")


def _stack_step(params, t, which, li):
    w = jnp.stack([params["gn"][br][t][which][li][0] for br in _BRANCHES])
    b = jnp.stack([params["gn"][br][t][which][li][1][None, :] for br in _BRANCHES])
    return w, b


def _head_params(params, br):
    out = []
    for (w, b) in params["out"][br]:
        out.append(w)
        out.append(b[None, :])
    return tuple(out)


def kernel(nodes, edges, coordinates, params, senders, receivers):
    N = nodes.shape[0]
    E = edges.shape[0]

    senders = senders.astype(jnp.int32)
    receivers = receivers.astype(jnp.int32)

    wn = jnp.concatenate(
        [params["emb"][br]["node"][0][0] for br in _BRANCHES], axis=1
    )
    wn = jnp.pad(wn, ((0, 1), (0, 0)))
    bn = jnp.concatenate(
        [params["emb"][br]["node"][0][1] for br in _BRANCHES]
    )[None, :]
    we = jnp.concatenate(
        [params["emb"][br]["edge"][0][0] for br in _BRANCHES], axis=1
    )
    be = jnp.concatenate(
        [params["emb"][br]["edge"][0][1] for br in _BRANCHES]
    )[None, :]

    nodes8 = jnp.pad(nodes, ((0, 0), (0, 1)))
    n3, n3_bf = _tc_embed(nodes8, wn, bn, 2000, with_bf16=True)
    e3 = _tc_embed(edges, we, be, 2000)

    gather192 = _make_sc_gather2(N, 192, E, jnp.bfloat16)
    scatter192 = _make_sc_scatter(N, 192, E, 32)
    zeros32 = jnp.zeros((N, 32), jnp.float32)

    for t in range(4):
        w1e, b1e = _stack_step(params, t, "edge", 0)
        w2e, b2e = _stack_step(params, t, "edge", 1)
        w1n, b1n = _stack_step(params, t, "node", 0)
        w2n, b2n = _stack_step(params, t, "node", 1)

        gs, gr = gather192(n3_bf, senders, receivers)
        e3 = _tc_edge_step(e3, gr, gs, w1e, b1e, w2e, b2e, 2000)
        agg = scatter192(e3, receivers, zeros32)
        if t < 3:
            n3, n3_bf = _tc_node_step(agg, n3, w1n, b1n, w2n, b2n, 2000,
                                      with_bf16=True)
        else:
            n3 = _tc_node_step(agg, n3, w1n, b1n, w2n, b2n, 2000)

    raw, acc = _tc_mono_head(n3[:, 0:64], *_head_params(params, "mono"), 2000)
    monopoles = _tc_mean_sub(raw, acc, 2000)

    H = jnp.concatenate(
        [n3[:, 64:192], coordinates, jnp.zeros((N, 13), jnp.float32)], axis=1
    )
    gather144 = _make_sc_gather2(N, 144, E)
    hs, hr = gather144(H, senders, receivers)
    ew = _tc_edge_head(hs, hr, edges,
                       _head_params(params, "dipo"),
                       _head_params(params, "quad"), 2000)
    scatter16 = _make_sc_scatter(N, 16, E, 8)
    zeros8 = jnp.zeros((N, 8), jnp.float32)
    agg16 = scatter16(ew, receivers, zeros8)

    dipoles = agg16[:, 0:3]
    quadrupoles = agg16[:, 3:12].reshape(N, 3, 3)
    return (monopoles, dipoles, quadrupoles)


# speedup vs baseline: 1.1655x; 1.1655x over previous
import functools

import jax
import jax.numpy as jnp
from jax import lax
from jax.experimental import pallas as pl
from jax.experimental.pallas import tpu as pltpu
from jax.experimental.pallas import tpu_sc as plsc

try:
    _info = plsc.get_sparse_core_info()
    NC, NS = int(_info.num_cores), int(_info.num_subcores)
except Exception:
    NC, NS = 2, 16
NW = NC * NS

_K = 128


def _mila(x):
    return x * jnp.tanh(jax.nn.softplus(x - 1.0))


@functools.lru_cache(maxsize=None)
def _make_sc_gather2(N, C, E, dtype, RK):
    RS = RK * _K
    n_super = E // RS
    n_pairs = (n_super + NW - 1) // NW
    n_pairs = (n_pairs + 1) // 2
    mesh = plsc.VectorSubcoreMesh(
        core_axis_name="c", subcore_axis_name="s", num_cores=NC, num_subcores=NS
    )
    out = jax.ShapeDtypeStruct((E, C), dtype)

    @functools.partial(
        pl.kernel,
        out_type=(out, out),
        mesh=mesh,
        compiler_params=pltpu.CompilerParams(use_tc_tiling_on_sc=False),
        scratch_types=[
            pltpu.VMEM((RK, _K), jnp.int32),
            pltpu.VMEM((RK, _K), jnp.int32),
            pltpu.VMEM((RK, _K), jnp.int32),
            pltpu.VMEM((RK, _K), jnp.int32),
            pltpu.VMEM((RS, C), dtype),
            pltpu.VMEM((RS, C), dtype),
            pltpu.VMEM((RS, C), dtype),
            pltpu.VMEM((RS, C), dtype),
            pltpu.SemaphoreType.DMA,
            pltpu.SemaphoreType.DMA,
            pltpu.SemaphoreType.DMA,
            pltpu.SemaphoreType.DMA,
            pltpu.SemaphoreType.DMA,
            pltpu.SemaphoreType.DMA,
        ],
    )
    def k(table, idx_s, idx_r, out_s, out_r,
          ia_s, ia_r, ib_s, ib_r, ra_s, ra_r, rb_s, rb_r,
          la, lb, ga, gb, oa, ob):
        wid = lax.axis_index("s") * NC + lax.axis_index("c")

        def issue_idx(sci, iv_s, iv_r, sem):
            pltpu.async_copy(idx_s.at[pl.ds(sci * RK, RK)], iv_s, sem)
            pltpu.async_copy(idx_r.at[pl.ds(sci * RK, RK)], iv_r, sem)

        def wait_idx(iv_s, iv_r, sem):
            pltpu.make_async_copy(idx_s.at[pl.ds(0, RK)], iv_s, sem).wait()
            pltpu.make_async_copy(idx_r.at[pl.ds(0, RK)], iv_r, sem).wait()

        def issue_gather(iv_s, iv_r, rv_s, rv_r, sem):
            for j in range(RK):
                pltpu.async_copy(
                    table.at[iv_s.at[j]], rv_s.at[pl.ds(j * _K, _K)], sem)
                pltpu.async_copy(
                    table.at[iv_r.at[j]], rv_r.at[pl.ds(j * _K, _K)], sem)

        def wait_gather(iv_s, iv_r, rv_s, rv_r, sem):
            for j in range(RK):
                pltpu.make_async_copy(
                    table.at[iv_s.at[j]], rv_s.at[pl.ds(j * _K, _K)], sem
                ).wait()
                pltpu.make_async_copy(
                    table.at[iv_r.at[j]], rv_r.at[pl.ds(j * _K, _K)], sem
                ).wait()

        def issue_out(sci, rv_s, rv_r, sem):
            pltpu.async_copy(rv_s, out_s.at[pl.ds(sci * RS, RS)], sem)
            pltpu.async_copy(rv_r, out_r.at[pl.ds(sci * RS, RS)], sem)

        def wait_out(rv_s, rv_r, sem):
            pltpu.make_async_copy(rv_s, out_s.at[pl.ds(0, RS)], sem).wait()
            pltpu.make_async_copy(rv_r, out_r.at[pl.ds(0, RS)], sem).wait()

        @pl.loop(0, n_pairs)
        def _(t):
            sa = wid + (2 * t) * NW
            sb = wid + (2 * t + 1) * NW
            va = sa < n_super
            vb = sb < n_super
            @pl.when(va)
            def _a0():
                issue_idx(sa, ia_s, ia_r, la)

            @pl.when(vb)
            def _b0():
                issue_idx(sb, ib_s, ib_r, lb)

            @pl.when(va)
            def _a1():
                wait_idx(ia_s, ia_r, la)
                issue_gather(ia_s, ia_r, ra_s, ra_r, ga)

            @pl.when(vb)
            def _b1():
                wait_idx(ib_s, ib_r, lb)
                issue_gather(ib_s, ib_r, rb_s, rb_r, gb)

            @pl.when(va)
            def _a2():
                wait_gather(ia_s, ia_r, ra_s, ra_r, ga)
                issue_out(sa, ra_s, ra_r, oa)

            @pl.when(vb)
            def _b2():
                wait_gather(ib_s, ib_r, rb_s, rb_r, gb)
                issue_out(sb, rb_s, rb_r, ob)

            @pl.when(va)
            def _a3():
                wait_out(ra_s, ra_r, oa)

            @pl.when(vb)
            def _b3():
                wait_out(rb_s, rb_r, ob)

    return k


@functools.lru_cache(maxsize=None)
def _make_sc_scatter(N, C, E, CW, RK):
    n_passes = C // (2 * CW)
    RS = RK * _K
    n_super = E // RS
    n_pairs = (n_super + NS - 1) // NS
    n_pairs = (n_pairs + 1) // 2
    rpt = N // NS
    mesh = plsc.VectorSubcoreMesh(
        core_axis_name="c", subcore_axis_name="s", num_cores=NC, num_subcores=NS
    )

    @functools.partial(
        pl.kernel,
        out_type=jax.ShapeDtypeStruct((N, C), jnp.float32),
        mesh=mesh,
        compiler_params=pltpu.CompilerParams(use_tc_tiling_on_sc=False),
        scratch_types=[
            pltpu.VMEM((RK, _K), jnp.int32),
            pltpu.VMEM((RK, _K), jnp.int32),
            pltpu.VMEM((RS, CW), jnp.float32),
            pltpu.VMEM((RS, CW), jnp.float32),
            pltpu.VMEM_SHARED((N, CW), jnp.float32),
            pltpu.SemaphoreType.DMA,
            pltpu.SemaphoreType.DMA,
            pltpu.SemaphoreType.DMA,
            pltpu.SemaphoreType.DMA,
        ],
    )
    def k(vals, idx, zeros, out, ia, ib, ra, rb, acc, la, lb, sa_sem, sb_sem):
        cid = lax.axis_index("c")
        sid = lax.axis_index("s")
        row0 = sid * rpt

        def issue_load(sci, iv, rv, sem, col0):
            pltpu.async_copy(idx.at[pl.ds(sci * RK, RK)], iv, sem)
            pltpu.async_copy(
                vals.at[pl.ds(sci * RS, RS), pl.ds(col0, CW)], rv, sem)

        def wait_load(iv, rv, sem, col0):
            pltpu.make_async_copy(idx.at[pl.ds(0, RK)], iv, sem).wait()
            pltpu.make_async_copy(
                vals.at[pl.ds(0, RS), pl.ds(col0, CW)], rv, sem).wait()

        def issue_add(iv, rv, sem):
            for j in range(RK):
                pltpu.async_copy(
                    rv.at[pl.ds(j * _K, _K)], acc.at[iv.at[j]], sem, add=True)

        def wait_add(iv, rv, sem):
            for j in range(RK):
                pltpu.make_async_copy(
                    rv.at[pl.ds(j * _K, _K)], acc.at[iv.at[j]], sem
                ).wait()

        for p in range(n_passes):
            col0 = (2 * p + cid) * CW
            pltpu.sync_copy(zeros.at[pl.ds(row0, rpt)], acc.at[pl.ds(row0, rpt)])
            plsc.subcore_barrier()

            @pl.loop(0, n_pairs)
            def _(t):
                sa = sid + (2 * t) * NS
                sb = sid + (2 * t + 1) * NS
                va = sa < n_super
                vb = sb < n_super

                @pl.when(va)
                def _a0():
                    issue_load(sa, ia, ra, la, col0)

                @pl.when(vb)
                def _b0():
                    issue_load(sb, ib, rb, lb, col0)

                @pl.when(va)
                def _a1():
                    wait_load(ia, ra, la, col0)
                    issue_add(ia, ra, sa_sem)

                @pl.when(vb)
                def _b1():
                    wait_load(ib, rb, lb, col0)
                    issue_add(ib, rb, sb_sem)

                @pl.when(va)
                def _a2():
                    wait_add(ia, ra, sa_sem)

                @pl.when(vb)
                def _b2():
                    wait_add(ib, rb, sb_sem)

            plsc.subcore_barrier()
            pltpu.sync_copy(
                acc.at[pl.ds(row0, rpt)],
                out.at[pl.ds(row0, rpt), pl.ds(col0, CW)],
            )
            plsc.subcore_barrier()

    return k


def _embed_body(x_ref, w_ref, b_ref, o_ref, obf_ref=None):
    o = _mila(
        jnp.dot(x_ref[...], w_ref[...], preferred_element_type=jnp.float32)
        + b_ref[...]
    )
    o_ref[...] = o
    if obf_ref is not None:
        obf_ref[...] = o.astype(jnp.bfloat16)


def _tc_embed(x, w, b, blk, with_bf16=False):
    R, Din = x.shape
    Dout = w.shape[1]
    out_specs = pl.BlockSpec((blk, Dout), lambda i: (i, 0))
    out_shape = jax.ShapeDtypeStruct((R, Dout), jnp.float32)
    if with_bf16:
        out_specs = [out_specs, pl.BlockSpec((blk, Dout), lambda i: (i, 0))]
        out_shape = [out_shape, jax.ShapeDtypeStruct((R, Dout), jnp.bfloat16)]
    return pl.pallas_call(
        _embed_body,
        grid=(R // blk,),
        in_specs=[
            pl.BlockSpec((blk, Din), lambda i: (i, 0)),
            pl.BlockSpec((Din, Dout), lambda i: (0, 0)),
            pl.BlockSpec((1, Dout), lambda i: (0, 0)),
        ],
        out_specs=out_specs,
        out_shape=out_shape,
    )(x, w, b)


def _edge_step_body(e_ref, gr_ref, gs_ref, w1_ref, b1_ref, w2_ref, b2_ref, o_ref):
    for br in range(3):
        c0 = br * 64
        x = jnp.concatenate(
            [
                e_ref[:, c0:c0 + 64],
                gr_ref[:, c0:c0 + 64].astype(jnp.float32),
                gs_ref[:, c0:c0 + 64].astype(jnp.float32),
            ],
            axis=1,
        )
        h = _mila(
            jnp.dot(x, w1_ref[br], preferred_element_type=jnp.float32)
            + b1_ref[br]
        )
        o_ref[:, c0:c0 + 64] = _mila(
            jnp.dot(h, w2_ref[br], preferred_element_type=jnp.float32)
            + b2_ref[br]
        )


def _tc_edge_step(e3, gr, gs, w1, b1, w2, b2, blk):
    E = e3.shape[0]
    return pl.pallas_call(
        _edge_step_body,
        grid=(E // blk,),
        in_specs=[
            pl.BlockSpec((blk, 192), lambda i: (i, 0)),
            pl.BlockSpec((blk, 192), lambda i: (i, 0)),
            pl.BlockSpec((blk, 192), lambda i: (i, 0)),
            pl.BlockSpec((3, 192, 64), lambda i: (0, 0, 0)),
            pl.BlockSpec((3, 1, 64), lambda i: (0, 0, 0)),
            pl.BlockSpec((3, 64, 64), lambda i: (0, 0, 0)),
            pl.BlockSpec((3, 1, 64), lambda i: (0, 0, 0)),
        ],
        out_specs=pl.BlockSpec((blk, 192), lambda i: (i, 0)),
        out_shape=jax.ShapeDtypeStruct((E, 192), jnp.float32),
    )(e3, gr, gs, w1, b1, w2, b2)


def _node_step_body(agg_ref, n_ref, w1_ref, b1_ref, w2_ref, b2_ref, o_ref,
                    obf_ref=None):
    for br in range(3):
        c0 = br * 64
        n_b = n_ref[:, c0:c0 + 64]
        x = jnp.concatenate([agg_ref[:, c0:c0 + 64], n_b], axis=1)
        h = _mila(
            jnp.dot(x, w1_ref[br], preferred_element_type=jnp.float32)
            + b1_ref[br]
        )
        o = (
            _mila(
                jnp.dot(h, w2_ref[br], preferred_element_type=jnp.float32)
                + b2_ref[br]
            )
            + n_b
        )
        o_ref[:, c0:c0 + 64] = o
        if obf_ref is not None:
            obf_ref[:, c0:c0 + 64] = o.astype(jnp.bfloat16)


def _tc_node_step(agg, n3, w1, b1, w2, b2, blk, with_bf16=False):
    N = n3.shape[0]
    out_specs = pl.BlockSpec((blk, 192), lambda i: (i, 0))
    out_shape = jax.ShapeDtypeStruct((N, 192), jnp.float32)
    if with_bf16:
        out_specs = [out_specs, pl.BlockSpec((blk, 192), lambda i: (i, 0))]
        out_shape = [out_shape, jax.ShapeDtypeStruct((N, 192), jnp.bfloat16)]
    return pl.pallas_call(
        _node_step_body,
        grid=(N // blk,),
        in_specs=[
            pl.BlockSpec((blk, 192), lambda i: (i, 0)),
            pl.BlockSpec((blk, 192), lambda i: (i, 0)),
            pl.BlockSpec((3, 128, 64), lambda i: (0, 0, 0)),
            pl.BlockSpec((3, 1, 64), lambda i: (0, 0, 0)),
            pl.BlockSpec((3, 64, 64), lambda i: (0, 0, 0)),
            pl.BlockSpec((3, 1, 64), lambda i: (0, 0, 0)),
        ],
        out_specs=out_specs,
        out_shape=out_shape,
    )(agg, n3, w1, b1, w2, b2)


def _mono_head_body(n_ref, w1_ref, b1_ref, w2_ref, b2_ref, w3_ref, b3_ref,
                    raw_ref, acc_ref):
    h = _mila(
        jnp.dot(n_ref[...], w1_ref[...], preferred_element_type=jnp.float32)
        + b1_ref[...]
    )
    h = _mila(
        jnp.dot(h, w2_ref[...], preferred_element_type=jnp.float32)
        + b2_ref[...]
    )
    raw = jnp.dot(h, w3_ref[...], preferred_element_type=jnp.float32) + b3_ref[...]
    raw_ref[...] = raw

    @pl.when(pl.program_id(0) == 0)
    def _():
        acc_ref[...] = jnp.zeros_like(acc_ref)

    acc_ref[...] += jnp.broadcast_to(jnp.sum(raw), acc_ref.shape)


def _tc_mono_head(n_mono, w1, b1, w2, b2, w3, b3, blk):
    N = n_mono.shape[0]
    return pl.pallas_call(
        _mono_head_body,
        grid=(N // blk,),
        in_specs=[
            pl.BlockSpec((blk, 64), lambda i: (i, 0)),
            pl.BlockSpec((64, 64), lambda i: (0, 0)),
            pl.BlockSpec((1, 64), lambda i: (0, 0)),
            pl.BlockSpec((64, 64), lambda i: (0, 0)),
            pl.BlockSpec((1, 64), lambda i: (0, 0)),
            pl.BlockSpec((64, 1), lambda i: (0, 0)),
            pl.BlockSpec((1, 1), lambda i: (0, 0)),
        ],
        out_specs=[
            pl.BlockSpec((blk, 1), lambda i: (i, 0)),
            pl.BlockSpec((8, 128), lambda i: (0, 0)),
        ],
        out_shape=[
            jax.ShapeDtypeStruct((N, 1), jnp.float32),
            jax.ShapeDtypeStruct((8, 128), jnp.float32),
        ],
    )(n_mono, w1, b1, w2, b2, w3, b3)


def _mean_sub_body(raw_ref, acc_ref, o_ref, *, count):
    total = jnp.sum(acc_ref[...]) / (8.0 * 128.0)
    o_ref[...] = raw_ref[...] - total / count


def _tc_mean_sub(raw, acc, blk):
    N = raw.shape[0]
    return pl.pallas_call(
        functools.partial(_mean_sub_body, count=float(N)),
        grid=(N // blk,),
        in_specs=[
            pl.BlockSpec((blk, 1), lambda i: (i, 0)),
            pl.BlockSpec((8, 128), lambda i: (0, 0)),
        ],
        out_specs=pl.BlockSpec((blk, 1), lambda i: (i, 0)),
        out_shape=jax.ShapeDtypeStruct((N, 1), jnp.float32),
    )(raw, acc)


def _edge_head_body(hs_ref, hr_ref, ed_ref,
                    w1d_ref, b1d_ref, w2d_ref, b2d_ref, w3d_ref, b3d_ref,
                    w1q_ref, b1q_ref, w2q_ref, b2q_ref, w3q_ref, b3q_ref,
                    o_ref):
    ed = ed_ref[...]

    def head(off, w1, b1, w2, b2, w3, b3):
        x = jnp.concatenate(
            [hs_ref[:, off:off + 64], hr_ref[:, off:off + 64], ed], axis=1
        )
        h = _mila(jnp.dot(x, w1[...], preferred_element_type=jnp.float32) + b1[...])
        h = _mila(jnp.dot(h, w2[...], preferred_element_type=jnp.float32) + b2[...])
        return jnp.dot(h, w3[...], preferred_element_type=jnp.float32) + b3[...]

    wd = head(0, w1d_ref, b1d_ref, w2d_ref, b2d_ref, w3d_ref, b3d_ref)
    wq = head(64, w1q_ref, b1q_ref, w2q_ref, b2q_ref, w3q_ref, b3q_ref)

    v = hs_ref[:, 128:131] - hr_ref[:, 128:131]
    vx, vy, vz = v[:, 0:1], v[:, 1:2], v[:, 2:3]
    tr3 = (vx * vx + vy * vy + vz * vz) * (1.0 / 3.0)
    q00 = wq * (vx * vx - tr3)
    q11 = wq * (vy * vy - tr3)
    q22 = wq * (vz * vz - tr3)
    q01 = wq * (vx * vy)
    q02 = wq * (vx * vz)
    q12 = wq * (vy * vz)
    zero = jnp.zeros_like(wd)
    o_ref[...] = jnp.concatenate(
        [wd * vx, wd * vy, wd * vz,
         q00, q01, q02, q01, q11, q12, q02, q12, q22,
         zero, zero, zero, zero],
        axis=1,
    )


def _tc_edge_head(hs, hr, edges, pd, pq, blk):
    E = hs.shape[0]
    w1d, b1d, w2d, b2d, w3d, b3d = pd
    w1q, b1q, w2q, b2q, w3q, b3q = pq
    wspec = lambda shp: pl.BlockSpec(shp, lambda i: (0, 0))
    return pl.pallas_call(
        _edge_head_body,
        grid=(E // blk,),
        in_specs=[
            pl.BlockSpec((blk, 144), lambda i: (i, 0)),
            pl.BlockSpec((blk, 144), lambda i: (i, 0)),
            pl.BlockSpec((blk, 32), lambda i: (i, 0)),
            wspec((160, 64)), wspec((1, 64)), wspec((64, 64)), wspec((1, 64)),
            wspec((64, 1)), wspec((1, 1)),
            wspec((160, 64)), wspec((1, 64)), wspec((64, 64)), wspec((1, 64)),
            wspec((64, 1)), wspec((1, 1)),
        ],
        out_specs=pl.BlockSpec((blk, 16), lambda i: (i, 0)),
        out_shape=jax.ShapeDtypeStruct((E, 16), jnp.float32),
    )(hs, hr, edges, w1d, b1d, w2d, b2d, w3d, b3d,
      w1q, b1q, w2q, b2q, w3q, b3q)


_BRANCHES = ("mono", "dipo", "quad")


def _stack_step(params, t, which, li):
    w = jnp.stack([params["gn"][br][t][which][li][0] for br in _BRANCHES])
    b = jnp.stack([params["gn"][br][t][which][li][1][None, :] for br in _BRANCHES])
    return w, b


def _head_params(params, br):
    out = []
    for (w, b) in params["out"][br]:
        out.append(w)
        out.append(b[None, :])
    return tuple(out)


def kernel(nodes, edges, coordinates, params, senders, receivers):
    N = nodes.shape[0]
    E = edges.shape[0]

    senders = senders.astype(jnp.int32)
    receivers = receivers.astype(jnp.int32)
    senders2 = senders.reshape(E // _K, _K)
    receivers2 = receivers.reshape(E // _K, _K)

    wn = jnp.concatenate(
        [params["emb"][br]["node"][0][0] for br in _BRANCHES], axis=1
    )
    wn = jnp.pad(wn, ((0, 1), (0, 0)))
    bn = jnp.concatenate(
        [params["emb"][br]["node"][0][1] for br in _BRANCHES]
    )[None, :]
    we = jnp.concatenate(
        [params["emb"][br]["edge"][0][0] for br in _BRANCHES], axis=1
    )
    be = jnp.concatenate(
        [params["emb"][br]["edge"][0][1] for br in _BRANCHES]
    )[None, :]

    nodes8 = jnp.pad(nodes, ((0, 0), (0, 1)))
    n3, n3_bf = _tc_embed(nodes8, wn, bn, 2000, with_bf16=True)
    e3 = _tc_embed(edges, we, be, 2000)

    gather192 = _make_sc_gather2(N, 192, E, jnp.bfloat16, 2)
    scatter192 = _make_sc_scatter(N, 192, E, 32, 2)
    zeros32 = jnp.zeros((N, 32), jnp.float32)

    for t in range(4):
        w1e, b1e = _stack_step(params, t, "edge", 0)
        w2e, b2e = _stack_step(params, t, "edge", 1)
        w1n, b1n = _stack_step(params, t, "node", 0)
        w2n, b2n = _stack_step(params, t, "node", 1)

        gs, gr = gather192(n3_bf, senders2, receivers2)
        e3 = _tc_edge_step(e3, gr, gs, w1e, b1e, w2e, b2e, 2000)
        agg = scatter192(e3, receivers2, zeros32)
        if t < 3:
            n3, n3_bf = _tc_node_step(agg, n3, w1n, b1n, w2n, b2n, 2000,
                                      with_bf16=True)
        else:
            n3 = _tc_node_step(agg, n3, w1n, b1n, w2n, b2n, 2000)

    raw, acc = _tc_mono_head(n3[:, 0:64], *_head_params(params, "mono"), 2000)
    monopoles = _tc_mean_sub(raw, acc, 2000)

    H = jnp.concatenate(
        [n3[:, 64:192], coordinates, jnp.zeros((N, 13), jnp.float32)], axis=1
    )
    gather144 = _make_sc_gather2(N, 144, E, jnp.float32, 1)
    hs, hr = gather144(H, senders2, receivers2)
    ew = _tc_edge_head(hs, hr, edges,
                       _head_params(params, "dipo"),
                       _head_params(params, "quad"), 2000)
    scatter16 = _make_sc_scatter(N, 16, E, 8, 5)
    zeros8 = jnp.zeros((N, 8), jnp.float32)
    agg16 = scatter16(ew, receivers2, zeros8)

    dipoles = agg16[:, 0:3]
    quadrupoles = agg16[:, 3:12].reshape(N, 3, 3)
    return (monopoles, dipoles, quadrupoles)


# speedup vs baseline: 1.3498x; 1.1582x over previous
import functools

import jax
import jax.numpy as jnp
from jax import lax
from jax.experimental import pallas as pl
from jax.experimental.pallas import tpu as pltpu
from jax.experimental.pallas import tpu_sc as plsc

try:
    _info = plsc.get_sparse_core_info()
    NC, NS = int(_info.num_cores), int(_info.num_subcores)
except Exception:
    NC, NS = 2, 16
NW = NC * NS

_K = 128


def _mila(x):
    return x * jnp.tanh(jax.nn.softplus(x - 1.0))


@functools.lru_cache(maxsize=None)
def _make_sc_gather2(N, C, E, dtype, RK):
    RS = RK * _K
    n_super = E // RS
    n_pairs = (n_super + NW - 1) // NW
    n_pairs = (n_pairs + 1) // 2
    mesh = plsc.VectorSubcoreMesh(
        core_axis_name="c", subcore_axis_name="s", num_cores=NC, num_subcores=NS
    )
    out = jax.ShapeDtypeStruct((E, C), dtype)

    @functools.partial(
        pl.kernel,
        out_type=(out, out),
        mesh=mesh,
        compiler_params=pltpu.CompilerParams(use_tc_tiling_on_sc=False),
        scratch_types=[
            pltpu.VMEM((RK, _K), jnp.int32),
            pltpu.VMEM((RK, _K), jnp.int32),
            pltpu.VMEM((RK, _K), jnp.int32),
            pltpu.VMEM((RK, _K), jnp.int32),
            pltpu.VMEM((RS, C), dtype),
            pltpu.VMEM((RS, C), dtype),
            pltpu.VMEM((RS, C), dtype),
            pltpu.VMEM((RS, C), dtype),
            pltpu.SemaphoreType.DMA,
            pltpu.SemaphoreType.DMA,
            pltpu.SemaphoreType.DMA,
            pltpu.SemaphoreType.DMA,
            pltpu.SemaphoreType.DMA,
            pltpu.SemaphoreType.DMA,
        ],
    )
    def k(table, idx_s, idx_r, out_s, out_r,
          ia_s, ia_r, ib_s, ib_r, ra_s, ra_r, rb_s, rb_r,
          la, lb, ga, gb, oa, ob):
        wid = lax.axis_index("s") * NC + lax.axis_index("c")

        def issue_idx(sci, iv_s, iv_r, sem):
            pltpu.async_copy(idx_s.at[pl.ds(sci * RK, RK)], iv_s, sem)
            pltpu.async_copy(idx_r.at[pl.ds(sci * RK, RK)], iv_r, sem)

        def wait_idx(iv_s, iv_r, sem):
            pltpu.make_async_copy(idx_s.at[pl.ds(0, RK)], iv_s, sem).wait()
            pltpu.make_async_copy(idx_r.at[pl.ds(0, RK)], iv_r, sem).wait()

        def issue_gather(iv_s, iv_r, rv_s, rv_r, sem):
            for j in range(RK):
                pltpu.async_copy(
                    table.at[iv_s.at[j]], rv_s.at[pl.ds(j * _K, _K)], sem)
                pltpu.async_copy(
                    table.at[iv_r.at[j]], rv_r.at[pl.ds(j * _K, _K)], sem)

        def wait_gather(iv_s, iv_r, rv_s, rv_r, sem):
            for j in range(RK):
                pltpu.make_async_copy(
                    table.at[iv_s.at[j]], rv_s.at[pl.ds(j * _K, _K)], sem
                ).wait()
                pltpu.make_async_copy(
                    table.at[iv_r.at[j]], rv_r.at[pl.ds(j * _K, _K)], sem
                ).wait()

        def issue_out(sci, rv_s, rv_r, sem):
            pltpu.async_copy(rv_s, out_s.at[pl.ds(sci * RS, RS)], sem)
            pltpu.async_copy(rv_r, out_r.at[pl.ds(sci * RS, RS)], sem)

        def wait_out(rv_s, rv_r, sem):
            pltpu.make_async_copy(rv_s, out_s.at[pl.ds(0, RS)], sem).wait()
            pltpu.make_async_copy(rv_r, out_r.at[pl.ds(0, RS)], sem).wait()

        @pl.loop(0, n_pairs)
        def _(t):
            sa = wid + (2 * t) * NW
            sb = wid + (2 * t + 1) * NW
            va = sa < n_super
            vb = sb < n_super
            @pl.when(va)
            def _a0():
                issue_idx(sa, ia_s, ia_r, la)

            @pl.when(vb)
            def _b0():
                issue_idx(sb, ib_s, ib_r, lb)

            @pl.when(va)
            def _a1():
                wait_idx(ia_s, ia_r, la)
                issue_gather(ia_s, ia_r, ra_s, ra_r, ga)

            @pl.when(vb)
            def _b1():
                wait_idx(ib_s, ib_r, lb)
                issue_gather(ib_s, ib_r, rb_s, rb_r, gb)

            @pl.when(va)
            def _a2():
                wait_gather(ia_s, ia_r, ra_s, ra_r, ga)
                issue_out(sa, ra_s, ra_r, oa)

            @pl.when(vb)
            def _b2():
                wait_gather(ib_s, ib_r, rb_s, rb_r, gb)
                issue_out(sb, rb_s, rb_r, ob)

            @pl.when(va)
            def _a3():
                wait_out(ra_s, ra_r, oa)

            @pl.when(vb)
            def _b3():
                wait_out(rb_s, rb_r, ob)

    return k


@functools.lru_cache(maxsize=None)
def _make_sc_scatter(N, C, E, CW, RK):
    n_passes = C // (2 * CW)
    RS = RK * _K
    n_super = E // RS
    n_pairs = (n_super + NS - 1) // NS
    n_pairs = (n_pairs + 1) // 2
    rpt = N // NS
    mesh = plsc.VectorSubcoreMesh(
        core_axis_name="c", subcore_axis_name="s", num_cores=NC, num_subcores=NS
    )

    @functools.partial(
        pl.kernel,
        out_type=jax.ShapeDtypeStruct((N, C), jnp.float32),
        mesh=mesh,
        compiler_params=pltpu.CompilerParams(use_tc_tiling_on_sc=False),
        scratch_types=[
            pltpu.VMEM((RK, _K), jnp.int32),
            pltpu.VMEM((RK, _K), jnp.int32),
            pltpu.VMEM((RS, CW), jnp.float32),
            pltpu.VMEM((RS, CW), jnp.float32),
            pltpu.VMEM_SHARED((N, CW), jnp.float32),
            pltpu.SemaphoreType.DMA,
            pltpu.SemaphoreType.DMA,
            pltpu.SemaphoreType.DMA,
            pltpu.SemaphoreType.DMA,
        ],
    )
    def k(vals, idx, zeros, out, ia, ib, ra, rb, acc, la, lb, sa_sem, sb_sem):
        cid = lax.axis_index("c")
        sid = lax.axis_index("s")
        row0 = sid * rpt

        def issue_load(sci, iv, rv, sem, col0):
            pltpu.async_copy(idx.at[pl.ds(sci * RK, RK)], iv, sem)
            pltpu.async_copy(
                vals.at[pl.ds(sci * RS, RS), pl.ds(col0, CW)], rv, sem)

        def wait_load(iv, rv, sem, col0):
            pltpu.make_async_copy(idx.at[pl.ds(0, RK)], iv, sem).wait()
            pltpu.make_async_copy(
                vals.at[pl.ds(0, RS), pl.ds(col0, CW)], rv, sem).wait()

        def issue_add(iv, rv, sem):
            for j in range(RK):
                pltpu.async_copy(
                    rv.at[pl.ds(j * _K, _K)], acc.at[iv.at[j]], sem, add=True)

        def wait_add(iv, rv, sem):
            for j in range(RK):
                pltpu.make_async_copy(
                    rv.at[pl.ds(j * _K, _K)], acc.at[iv.at[j]], sem
                ).wait()

        for p in range(n_passes):
            col0 = (2 * p + cid) * CW
            pltpu.sync_copy(zeros.at[pl.ds(row0, rpt)], acc.at[pl.ds(row0, rpt)])
            plsc.subcore_barrier()

            @pl.loop(0, n_pairs)
            def _(t):
                sa = sid + (2 * t) * NS
                sb = sid + (2 * t + 1) * NS
                va = sa < n_super
                vb = sb < n_super

                @pl.when(va)
                def _a0():
                    issue_load(sa, ia, ra, la, col0)

                @pl.when(vb)
                def _b0():
                    issue_load(sb, ib, rb, lb, col0)

                @pl.when(va)
                def _a1():
                    wait_load(ia, ra, la, col0)
                    issue_add(ia, ra, sa_sem)

                @pl.when(vb)
                def _b1():
                    wait_load(ib, rb, lb, col0)
                    issue_add(ib, rb, sb_sem)

                @pl.when(va)
                def _a2():
                    wait_add(ia, ra, sa_sem)

                @pl.when(vb)
                def _b2():
                    wait_add(ib, rb, sb_sem)

            plsc.subcore_barrier()
            pltpu.sync_copy(
                acc.at[pl.ds(row0, rpt)],
                out.at[pl.ds(row0, rpt), pl.ds(col0, CW)],
            )
            plsc.subcore_barrier()

    return k


def _embed_body(x_ref, w_ref, b_ref, o_ref, obf_ref=None):
    o = _mila(
        jnp.dot(x_ref[...], w_ref[...], preferred_element_type=jnp.float32)
        + b_ref[...]
    )
    o_ref[...] = o
    if obf_ref is not None:
        obf_ref[...] = o.astype(jnp.bfloat16)


def _tc_embed(x, w, b, blk, with_bf16=False):
    R, Din = x.shape
    Dout = w.shape[1]
    out_specs = pl.BlockSpec((blk, Dout), lambda i: (i, 0))
    out_shape = jax.ShapeDtypeStruct((R, Dout), jnp.float32)
    if with_bf16:
        out_specs = [out_specs, pl.BlockSpec((blk, Dout), lambda i: (i, 0))]
        out_shape = [out_shape, jax.ShapeDtypeStruct((R, Dout), jnp.bfloat16)]
    return pl.pallas_call(
        _embed_body,
        grid=(R // blk,),
        in_specs=[
            pl.BlockSpec((blk, Din), lambda i: (i, 0)),
            pl.BlockSpec((Din, Dout), lambda i: (0, 0)),
            pl.BlockSpec((1, Dout), lambda i: (0, 0)),
        ],
        out_specs=out_specs,
        out_shape=out_shape,
    )(x, w, b)


def _edge_step_body(e_ref, gr_ref, gs_ref, w1_ref, b1_ref, w2_ref, b2_ref, o_ref):
    for br in range(3):
        c0 = br * 64
        x = jnp.concatenate(
            [
                e_ref[:, c0:c0 + 64],
                gr_ref[:, c0:c0 + 64].astype(jnp.float32),
                gs_ref[:, c0:c0 + 64].astype(jnp.float32),
            ],
            axis=1,
        )
        h = _mila(
            jnp.dot(x, w1_ref[br], preferred_element_type=jnp.float32)
            + b1_ref[br]
        )
        o_ref[:, c0:c0 + 64] = _mila(
            jnp.dot(h, w2_ref[br], preferred_element_type=jnp.float32)
            + b2_ref[br]
        )


def _tc_edge_step(e3, gr, gs, w1, b1, w2, b2, blk):
    E = e3.shape[0]
    return pl.pallas_call(
        _edge_step_body,
        grid=(E // blk,),
        in_specs=[
            pl.BlockSpec((blk, 192), lambda i: (i, 0)),
            pl.BlockSpec((blk, 192), lambda i: (i, 0)),
            pl.BlockSpec((blk, 192), lambda i: (i, 0)),
            pl.BlockSpec((3, 192, 64), lambda i: (0, 0, 0)),
            pl.BlockSpec((3, 1, 64), lambda i: (0, 0, 0)),
            pl.BlockSpec((3, 64, 64), lambda i: (0, 0, 0)),
            pl.BlockSpec((3, 1, 64), lambda i: (0, 0, 0)),
        ],
        out_specs=pl.BlockSpec((blk, 192), lambda i: (i, 0)),
        out_shape=jax.ShapeDtypeStruct((E, 192), jnp.float32),
    )(e3, gr, gs, w1, b1, w2, b2)


def _node_step_body(agg_ref, agg2_ref, n_ref, w1_ref, b1_ref, w2_ref, b2_ref,
                    o_ref, obf_ref=None):
    for br in range(3):
        c0 = br * 64
        n_b = n_ref[:, c0:c0 + 64]
        agg_b = agg_ref[:, c0:c0 + 64]
        if agg2_ref is not None:
            agg_b = agg_b + agg2_ref[:, c0:c0 + 64]
        x = jnp.concatenate([agg_b, n_b], axis=1)
        h = _mila(
            jnp.dot(x, w1_ref[br], preferred_element_type=jnp.float32)
            + b1_ref[br]
        )
        o = (
            _mila(
                jnp.dot(h, w2_ref[br], preferred_element_type=jnp.float32)
                + b2_ref[br]
            )
            + n_b
        )
        o_ref[:, c0:c0 + 64] = o
        if obf_ref is not None:
            obf_ref[:, c0:c0 + 64] = o.astype(jnp.bfloat16)


def _tc_node_step(agg, agg2, n3, w1, b1, w2, b2, blk):
    N = n3.shape[0]
    row_spec = pl.BlockSpec((blk, 192), lambda i: (i, 0))
    w_specs = [
        pl.BlockSpec((3, 128, 64), lambda i: (0, 0, 0)),
        pl.BlockSpec((3, 1, 64), lambda i: (0, 0, 0)),
        pl.BlockSpec((3, 64, 64), lambda i: (0, 0, 0)),
        pl.BlockSpec((3, 1, 64), lambda i: (0, 0, 0)),
    ]
    if agg2 is None:
        def body(agg_ref, n_ref, w1_ref, b1_ref, w2_ref, b2_ref, o_ref):
            _node_step_body(agg_ref, None, n_ref, w1_ref, b1_ref,
                            w2_ref, b2_ref, o_ref)
        in_specs = [row_spec, row_spec] + w_specs
        args = (agg, n3, w1, b1, w2, b2)
    else:
        body = _node_step_body
        in_specs = [row_spec, row_spec, row_spec] + w_specs
        args = (agg, agg2, n3, w1, b1, w2, b2)
    return pl.pallas_call(
        body,
        grid=(N // blk,),
        in_specs=in_specs,
        out_specs=pl.BlockSpec((blk, 192), lambda i: (i, 0)),
        out_shape=jax.ShapeDtypeStruct((N, 192), jnp.float32),
    )(*args)


def _mono_head_body(n_ref, w1_ref, b1_ref, w2_ref, b2_ref, w3_ref, b3_ref,
                    raw_ref, acc_ref):
    h = _mila(
        jnp.dot(n_ref[...], w1_ref[...], preferred_element_type=jnp.float32)
        + b1_ref[...]
    )
    h = _mila(
        jnp.dot(h, w2_ref[...], preferred_element_type=jnp.float32)
        + b2_ref[...]
    )
    raw = jnp.dot(h, w3_ref[...], preferred_element_type=jnp.float32) + b3_ref[...]
    raw_ref[...] = raw

    @pl.when(pl.program_id(0) == 0)
    def _():
        acc_ref[...] = jnp.zeros_like(acc_ref)

    acc_ref[...] += jnp.broadcast_to(jnp.sum(raw), acc_ref.shape)


def _tc_mono_head(n_mono, w1, b1, w2, b2, w3, b3, blk):
    N = n_mono.shape[0]
    return pl.pallas_call(
        _mono_head_body,
        grid=(N // blk,),
        in_specs=[
            pl.BlockSpec((blk, 64), lambda i: (i, 0)),
            pl.BlockSpec((64, 64), lambda i: (0, 0)),
            pl.BlockSpec((1, 64), lambda i: (0, 0)),
            pl.BlockSpec((64, 64), lambda i: (0, 0)),
            pl.BlockSpec((1, 64), lambda i: (0, 0)),
            pl.BlockSpec((64, 1), lambda i: (0, 0)),
            pl.BlockSpec((1, 1), lambda i: (0, 0)),
        ],
        out_specs=[
            pl.BlockSpec((blk, 1), lambda i: (i, 0)),
            pl.BlockSpec((8, 128), lambda i: (0, 0)),
        ],
        out_shape=[
            jax.ShapeDtypeStruct((N, 1), jnp.float32),
            jax.ShapeDtypeStruct((8, 128), jnp.float32),
        ],
    )(n_mono, w1, b1, w2, b2, w3, b3)


def _mean_sub_body(raw_ref, acc_ref, o_ref, *, count):
    total = jnp.sum(acc_ref[...]) / (8.0 * 128.0)
    o_ref[...] = raw_ref[...] - total / count


def _tc_mean_sub(raw, acc, blk):
    N = raw.shape[0]
    return pl.pallas_call(
        functools.partial(_mean_sub_body, count=float(N)),
        grid=(N // blk,),
        in_specs=[
            pl.BlockSpec((blk, 1), lambda i: (i, 0)),
            pl.BlockSpec((8, 128), lambda i: (0, 0)),
        ],
        out_specs=pl.BlockSpec((blk, 1), lambda i: (i, 0)),
        out_shape=jax.ShapeDtypeStruct((N, 1), jnp.float32),
    )(raw, acc)


def _edge_head_body(hs_ref, hr_ref, ed_ref,
                    w1d_ref, b1d_ref, w2d_ref, b2d_ref, w3d_ref, b3d_ref,
                    w1q_ref, b1q_ref, w2q_ref, b2q_ref, w3q_ref, b3q_ref,
                    o_ref):
    ed = ed_ref[...]

    def head(off, w1, b1, w2, b2, w3, b3):
        x = jnp.concatenate(
            [hs_ref[:, off:off + 64], hr_ref[:, off:off + 64], ed], axis=1
        )
        h = _mila(jnp.dot(x, w1[...], preferred_element_type=jnp.float32) + b1[...])
        h = _mila(jnp.dot(h, w2[...], preferred_element_type=jnp.float32) + b2[...])
        return jnp.dot(h, w3[...], preferred_element_type=jnp.float32) + b3[...]

    wd = head(0, w1d_ref, b1d_ref, w2d_ref, b2d_ref, w3d_ref, b3d_ref)
    wq = head(64, w1q_ref, b1q_ref, w2q_ref, b2q_ref, w3q_ref, b3q_ref)

    v = hs_ref[:, 128:131] - hr_ref[:, 128:131]
    vx, vy, vz = v[:, 0:1], v[:, 1:2], v[:, 2:3]
    tr3 = (vx * vx + vy * vy + vz * vz) * (1.0 / 3.0)
    q00 = wq * (vx * vx - tr3)
    q11 = wq * (vy * vy - tr3)
    q22 = wq * (vz * vz - tr3)
    q01 = wq * (vx * vy)
    q02 = wq * (vx * vz)
    q12 = wq * (vy * vz)
    zero = jnp.zeros_like(wd)
    o_ref[...] = jnp.concatenate(
        [wd * vx, wd * vy, wd * vz,
         q00, q01, q02, q01, q11, q12, q02, q12, q22,
         zero, zero, zero, zero],
        axis=1,
    )


def _tc_edge_head(hs, hr, edges, pd, pq, blk):
    E = hs.shape[0]
    w1d, b1d, w2d, b2d, w3d, b3d = pd
    w1q, b1q, w2q, b2q, w3q, b3q = pq
    wspec = lambda shp: pl.BlockSpec(shp, lambda i: (0, 0))
    return pl.pallas_call(
        _edge_head_body,
        grid=(E // blk,),
        in_specs=[
            pl.BlockSpec((blk, 144), lambda i: (i, 0)),
            pl.BlockSpec((blk, 144), lambda i: (i, 0)),
            pl.BlockSpec((blk, 32), lambda i: (i, 0)),
            wspec((160, 64)), wspec((1, 64)), wspec((64, 64)), wspec((1, 64)),
            wspec((64, 1)), wspec((1, 1)),
            wspec((160, 64)), wspec((1, 64)), wspec((64, 64)), wspec((1, 64)),
            wspec((64, 1)), wspec((1, 1)),
        ],
        out_specs=pl.BlockSpec((blk, 16), lambda i: (i, 0)),
        out_shape=jax.ShapeDtypeStruct((E, 16), jnp.float32),
    )(hs, hr, edges, w1d, b1d, w2d, b2d, w3d, b3d,
      w1q, b1q, w2q, b2q, w3q, b3q)


_BRANCHES = ("mono", "dipo", "quad")


def _stack_step(params, t, which, li):
    w = jnp.stack([params["gn"][br][t][which][li][0] for br in _BRANCHES])
    b = jnp.stack([params["gn"][br][t][which][li][1][None, :] for br in _BRANCHES])
    return w, b


def _head_params(params, br):
    out = []
    for (w, b) in params["out"][br]:
        out.append(w)
        out.append(b[None, :])
    return tuple(out)


def kernel(nodes, edges, coordinates, params, senders, receivers):
    N = nodes.shape[0]
    E = edges.shape[0]

    senders = senders.astype(jnp.int32)
    receivers = receivers.astype(jnp.int32)
    senders2 = senders.reshape(E // _K, _K)
    receivers2 = receivers.reshape(E // _K, _K)

    wn = jnp.concatenate(
        [params["emb"][br]["node"][0][0] for br in _BRANCHES], axis=1
    )
    wn = jnp.pad(wn, ((0, 1), (0, 0)))
    bn = jnp.concatenate(
        [params["emb"][br]["node"][0][1] for br in _BRANCHES]
    )[None, :]
    we = jnp.concatenate(
        [params["emb"][br]["edge"][0][0] for br in _BRANCHES], axis=1
    )
    be = jnp.concatenate(
        [params["emb"][br]["edge"][0][1] for br in _BRANCHES]
    )[None, :]

    nodes8 = jnp.pad(nodes, ((0, 0), (0, 1)))
    n3 = _tc_embed(nodes8, wn, bn, 2000)
    e3 = _tc_embed(edges, we, be, 2000)

    gather192 = _make_sc_gather2(N, 192, E, jnp.float32, 1)
    scatter192 = _make_sc_scatter(N, 192, E, 32, 2)
    zeros32 = jnp.zeros((N, 32), jnp.float32)

    for t in range(4):
        w1e, b1e = _stack_step(params, t, "edge", 0)
        w2e, b2e = _stack_step(params, t, "edge", 1)
        w1n, b1n = _stack_step(params, t, "node", 0)
        w2n, b2n = _stack_step(params, t, "node", 1)

        gs, gr = gather192(n3, senders2, receivers2)
        e3 = _tc_edge_step(e3, gr, gs, w1e, b1e, w2e, b2e, 2000)
        agg = scatter192(e3, receivers2, zeros32)
        n3 = _tc_node_step(agg, None, n3, w1n, b1n, w2n, b2n, 2000)

    raw, acc = _tc_mono_head(n3[:, 0:64], *_head_params(params, "mono"), 2000)
    monopoles = _tc_mean_sub(raw, acc, 2000)

    H = jnp.concatenate(
        [n3[:, 64:192], coordinates, jnp.zeros((N, 13), jnp.float32)], axis=1
    )
    gather144 = _make_sc_gather2(N, 144, E, jnp.float32, 1)
    hs, hr = gather144(H, senders2, receivers2)
    ew = _tc_edge_head(hs, hr, edges,
                       _head_params(params, "dipo"),
                       _head_params(params, "quad"), 2000)
    scatter16 = _make_sc_scatter(N, 16, E, 8, 5)
    zeros8 = jnp.zeros((N, 8), jnp.float32)
    agg16 = scatter16(ew, receivers2, zeros8)

    dipoles = agg16[:, 0:3]
    quadrupoles = agg16[:, 3:12].reshape(N, 3, 3)
    return (monopoles, dipoles, quadrupoles)


# speedup vs baseline: 1.3608x; 1.0081x over previous
import functools

import jax
import jax.numpy as jnp
from jax import lax
from jax.experimental import pallas as pl
from jax.experimental.pallas import tpu as pltpu
from jax.experimental.pallas import tpu_sc as plsc

try:
    _info = plsc.get_sparse_core_info()
    NC, NS = int(_info.num_cores), int(_info.num_subcores)
except Exception:
    NC, NS = 2, 16
NW = NC * NS

_K = 128


def _mila(x):
    return x * jnp.tanh(jax.nn.softplus(x - 1.0))


@functools.lru_cache(maxsize=None)
def _make_sc_gather2(N, C, E, dtype, RK):
    RS = RK * _K
    n_super = E // RS
    n_pairs = (n_super + NW - 1) // NW
    n_pairs = (n_pairs + 1) // 2
    mesh = plsc.VectorSubcoreMesh(
        core_axis_name="c", subcore_axis_name="s", num_cores=NC, num_subcores=NS
    )
    out = jax.ShapeDtypeStruct((E, C), dtype)

    @functools.partial(
        pl.kernel,
        out_type=(out, out),
        mesh=mesh,
        compiler_params=pltpu.CompilerParams(use_tc_tiling_on_sc=False),
        scratch_types=[
            pltpu.VMEM((RK, _K), jnp.int32),
            pltpu.VMEM((RK, _K), jnp.int32),
            pltpu.VMEM((RK, _K), jnp.int32),
            pltpu.VMEM((RK, _K), jnp.int32),
            pltpu.VMEM((RS, C), dtype),
            pltpu.VMEM((RS, C), dtype),
            pltpu.VMEM((RS, C), dtype),
            pltpu.VMEM((RS, C), dtype),
            pltpu.SemaphoreType.DMA,
            pltpu.SemaphoreType.DMA,
            pltpu.SemaphoreType.DMA,
            pltpu.SemaphoreType.DMA,
            pltpu.SemaphoreType.DMA,
            pltpu.SemaphoreType.DMA,
        ],
    )
    def k(table, idx_s, idx_r, out_s, out_r,
          ia_s, ia_r, ib_s, ib_r, ra_s, ra_r, rb_s, rb_r,
          la, lb, ga, gb, oa, ob):
        wid = lax.axis_index("s") * NC + lax.axis_index("c")

        def issue_idx(sci, iv_s, iv_r, sem):
            pltpu.async_copy(idx_s.at[pl.ds(sci * RK, RK)], iv_s, sem)
            pltpu.async_copy(idx_r.at[pl.ds(sci * RK, RK)], iv_r, sem)

        def wait_idx(iv_s, iv_r, sem):
            pltpu.make_async_copy(idx_s.at[pl.ds(0, RK)], iv_s, sem).wait()
            pltpu.make_async_copy(idx_r.at[pl.ds(0, RK)], iv_r, sem).wait()

        def issue_gather(iv_s, iv_r, rv_s, rv_r, sem):
            for j in range(RK):
                pltpu.async_copy(
                    table.at[iv_s.at[j]], rv_s.at[pl.ds(j * _K, _K)], sem)
                pltpu.async_copy(
                    table.at[iv_r.at[j]], rv_r.at[pl.ds(j * _K, _K)], sem)

        def wait_gather(iv_s, iv_r, rv_s, rv_r, sem):
            for j in range(RK):
                pltpu.make_async_copy(
                    table.at[iv_s.at[j]], rv_s.at[pl.ds(j * _K, _K)], sem
                ).wait()
                pltpu.make_async_copy(
                    table.at[iv_r.at[j]], rv_r.at[pl.ds(j * _K, _K)], sem
                ).wait()

        def issue_out(sci, rv_s, rv_r, sem):
            pltpu.async_copy(rv_s, out_s.at[pl.ds(sci * RS, RS)], sem)
            pltpu.async_copy(rv_r, out_r.at[pl.ds(sci * RS, RS)], sem)

        def wait_out(rv_s, rv_r, sem):
            pltpu.make_async_copy(rv_s, out_s.at[pl.ds(0, RS)], sem).wait()
            pltpu.make_async_copy(rv_r, out_r.at[pl.ds(0, RS)], sem).wait()

        @pl.loop(0, n_pairs)
        def _(t):
            sa = wid + (2 * t) * NW
            sb = wid + (2 * t + 1) * NW
            va = sa < n_super
            vb = sb < n_super
            @pl.when(va)
            def _a0():
                issue_idx(sa, ia_s, ia_r, la)

            @pl.when(vb)
            def _b0():
                issue_idx(sb, ib_s, ib_r, lb)

            @pl.when(va)
            def _a1():
                wait_idx(ia_s, ia_r, la)
                issue_gather(ia_s, ia_r, ra_s, ra_r, ga)

            @pl.when(vb)
            def _b1():
                wait_idx(ib_s, ib_r, lb)
                issue_gather(ib_s, ib_r, rb_s, rb_r, gb)

            @pl.when(va)
            def _a2():
                wait_gather(ia_s, ia_r, ra_s, ra_r, ga)
                issue_out(sa, ra_s, ra_r, oa)

            @pl.when(vb)
            def _b2():
                wait_gather(ib_s, ib_r, rb_s, rb_r, gb)
                issue_out(sb, rb_s, rb_r, ob)

            @pl.when(va)
            def _a3():
                wait_out(ra_s, ra_r, oa)

            @pl.when(vb)
            def _b3():
                wait_out(rb_s, rb_r, ob)

    return k


@functools.lru_cache(maxsize=None)
def _make_sc_scatter(N, C, E, CW, RK):
    n_passes = C // (2 * CW)
    RS = RK * _K
    n_super = E // RS
    n_pairs = (n_super + NS - 1) // NS
    n_pairs = (n_pairs + 1) // 2
    rpt = N // NS
    mesh = plsc.VectorSubcoreMesh(
        core_axis_name="c", subcore_axis_name="s", num_cores=NC, num_subcores=NS
    )

    @functools.partial(
        pl.kernel,
        out_type=jax.ShapeDtypeStruct((N, C), jnp.float32),
        mesh=mesh,
        compiler_params=pltpu.CompilerParams(use_tc_tiling_on_sc=False),
        scratch_types=[
            pltpu.VMEM((RK, _K), jnp.int32),
            pltpu.VMEM((RK, _K), jnp.int32),
            pltpu.VMEM((RS, CW), jnp.float32),
            pltpu.VMEM((RS, CW), jnp.float32),
            pltpu.VMEM_SHARED((N, CW), jnp.float32),
            pltpu.SemaphoreType.DMA,
            pltpu.SemaphoreType.DMA,
            pltpu.SemaphoreType.DMA,
            pltpu.SemaphoreType.DMA,
        ],
    )
    def k(vals, idx, zeros, out, ia, ib, ra, rb, acc, la, lb, sa_sem, sb_sem):
        cid = lax.axis_index("c")
        sid = lax.axis_index("s")
        row0 = sid * rpt

        def issue_load(sci, iv, rv, sem, col0):
            pltpu.async_copy(idx.at[pl.ds(sci * RK, RK)], iv, sem)
            pltpu.async_copy(
                vals.at[pl.ds(sci * RS, RS), pl.ds(col0, CW)], rv, sem)

        def wait_load(iv, rv, sem, col0):
            pltpu.make_async_copy(idx.at[pl.ds(0, RK)], iv, sem).wait()
            pltpu.make_async_copy(
                vals.at[pl.ds(0, RS), pl.ds(col0, CW)], rv, sem).wait()

        def issue_add(iv, rv, sem):
            for j in range(RK):
                pltpu.async_copy(
                    rv.at[pl.ds(j * _K, _K)], acc.at[iv.at[j]], sem, add=True)

        def wait_add(iv, rv, sem):
            for j in range(RK):
                pltpu.make_async_copy(
                    rv.at[pl.ds(j * _K, _K)], acc.at[iv.at[j]], sem
                ).wait()

        for p in range(n_passes):
            col0 = (2 * p + cid) * CW
            pltpu.sync_copy(zeros.at[pl.ds(row0, rpt)], acc.at[pl.ds(row0, rpt)])
            plsc.subcore_barrier()

            @pl.loop(0, n_pairs)
            def _(t):
                sa = sid + (2 * t) * NS
                sb = sid + (2 * t + 1) * NS
                va = sa < n_super
                vb = sb < n_super

                @pl.when(va)
                def _a0():
                    issue_load(sa, ia, ra, la, col0)

                @pl.when(vb)
                def _b0():
                    issue_load(sb, ib, rb, lb, col0)

                @pl.when(va)
                def _a1():
                    wait_load(ia, ra, la, col0)
                    issue_add(ia, ra, sa_sem)

                @pl.when(vb)
                def _b1():
                    wait_load(ib, rb, lb, col0)
                    issue_add(ib, rb, sb_sem)

                @pl.when(va)
                def _a2():
                    wait_add(ia, ra, sa_sem)

                @pl.when(vb)
                def _b2():
                    wait_add(ib, rb, sb_sem)

            plsc.subcore_barrier()
            pltpu.sync_copy(
                acc.at[pl.ds(row0, rpt)],
                out.at[pl.ds(row0, rpt), pl.ds(col0, CW)],
            )
            plsc.subcore_barrier()

    return k


def _embed_body(x_ref, w_ref, b_ref, o_ref, obf_ref=None):
    o = _mila(
        jnp.dot(x_ref[...], w_ref[...], preferred_element_type=jnp.float32)
        + b_ref[...]
    )
    o_ref[...] = o
    if obf_ref is not None:
        obf_ref[...] = o.astype(jnp.bfloat16)


def _tc_embed(x, w, b, blk, with_bf16=False):
    R, Din = x.shape
    Dout = w.shape[1]
    out_specs = pl.BlockSpec((blk, Dout), lambda i: (i, 0))
    out_shape = jax.ShapeDtypeStruct((R, Dout), jnp.float32)
    if with_bf16:
        out_specs = [out_specs, pl.BlockSpec((blk, Dout), lambda i: (i, 0))]
        out_shape = [out_shape, jax.ShapeDtypeStruct((R, Dout), jnp.bfloat16)]
    return pl.pallas_call(
        _embed_body,
        grid=(R // blk,),
        in_specs=[
            pl.BlockSpec((blk, Din), lambda i: (i, 0)),
            pl.BlockSpec((Din, Dout), lambda i: (0, 0)),
            pl.BlockSpec((1, Dout), lambda i: (0, 0)),
        ],
        out_specs=out_specs,
        out_shape=out_shape,
    )(x, w, b)


def _edge_step_body(e_ref, gr_ref, gs_ref, w1_ref, b1_ref, w2_ref, b2_ref, o_ref):
    for br in range(3):
        c0 = br * 64
        x = jnp.concatenate(
            [
                e_ref[:, c0:c0 + 64],
                gr_ref[:, c0:c0 + 64].astype(jnp.float32),
                gs_ref[:, c0:c0 + 64].astype(jnp.float32),
            ],
            axis=1,
        )
        h = _mila(
            jnp.dot(x, w1_ref[br], preferred_element_type=jnp.float32)
            + b1_ref[br]
        )
        o_ref[:, c0:c0 + 64] = _mila(
            jnp.dot(h, w2_ref[br], preferred_element_type=jnp.float32)
            + b2_ref[br]
        )


def _tc_edge_step(e3, gr, gs, w1, b1, w2, b2, blk):
    E = e3.shape[0]
    return pl.pallas_call(
        _edge_step_body,
        grid=(E // blk,),
        in_specs=[
            pl.BlockSpec((blk, 192), lambda i: (i, 0)),
            pl.BlockSpec((blk, 192), lambda i: (i, 0)),
            pl.BlockSpec((blk, 192), lambda i: (i, 0)),
            pl.BlockSpec((3, 192, 64), lambda i: (0, 0, 0)),
            pl.BlockSpec((3, 1, 64), lambda i: (0, 0, 0)),
            pl.BlockSpec((3, 64, 64), lambda i: (0, 0, 0)),
            pl.BlockSpec((3, 1, 64), lambda i: (0, 0, 0)),
        ],
        out_specs=pl.BlockSpec((blk, 192), lambda i: (i, 0)),
        out_shape=jax.ShapeDtypeStruct((E, 192), jnp.float32),
    )(e3, gr, gs, w1, b1, w2, b2)


def _node_step_body(agg_ref, agg2_ref, n_ref, w1_ref, b1_ref, w2_ref, b2_ref,
                    o_ref, obf_ref=None):
    for br in range(3):
        c0 = br * 64
        n_b = n_ref[:, c0:c0 + 64]
        agg_b = agg_ref[:, c0:c0 + 64]
        if agg2_ref is not None:
            agg_b = agg_b + agg2_ref[:, c0:c0 + 64]
        x = jnp.concatenate([agg_b, n_b], axis=1)
        h = _mila(
            jnp.dot(x, w1_ref[br], preferred_element_type=jnp.float32)
            + b1_ref[br]
        )
        o = (
            _mila(
                jnp.dot(h, w2_ref[br], preferred_element_type=jnp.float32)
                + b2_ref[br]
            )
            + n_b
        )
        o_ref[:, c0:c0 + 64] = o
        if obf_ref is not None:
            obf_ref[:, c0:c0 + 64] = o.astype(jnp.bfloat16)


def _tc_node_step(agg, agg2, n3, w1, b1, w2, b2, blk):
    N = n3.shape[0]
    row_spec = pl.BlockSpec((blk, 192), lambda i: (i, 0))
    w_specs = [
        pl.BlockSpec((3, 128, 64), lambda i: (0, 0, 0)),
        pl.BlockSpec((3, 1, 64), lambda i: (0, 0, 0)),
        pl.BlockSpec((3, 64, 64), lambda i: (0, 0, 0)),
        pl.BlockSpec((3, 1, 64), lambda i: (0, 0, 0)),
    ]
    if agg2 is None:
        def body(agg_ref, n_ref, w1_ref, b1_ref, w2_ref, b2_ref, o_ref):
            _node_step_body(agg_ref, None, n_ref, w1_ref, b1_ref,
                            w2_ref, b2_ref, o_ref)
        in_specs = [row_spec, row_spec] + w_specs
        args = (agg, n3, w1, b1, w2, b2)
    else:
        body = _node_step_body
        in_specs = [row_spec, row_spec, row_spec] + w_specs
        args = (agg, agg2, n3, w1, b1, w2, b2)
    return pl.pallas_call(
        body,
        grid=(N // blk,),
        in_specs=in_specs,
        out_specs=pl.BlockSpec((blk, 192), lambda i: (i, 0)),
        out_shape=jax.ShapeDtypeStruct((N, 192), jnp.float32),
    )(*args)


def _mono_head_body(n_ref, w1_ref, b1_ref, w2_ref, b2_ref, w3_ref, b3_ref,
                    raw_ref, acc_ref):
    h = _mila(
        jnp.dot(n_ref[...], w1_ref[...], preferred_element_type=jnp.float32)
        + b1_ref[...]
    )
    h = _mila(
        jnp.dot(h, w2_ref[...], preferred_element_type=jnp.float32)
        + b2_ref[...]
    )
    raw = jnp.dot(h, w3_ref[...], preferred_element_type=jnp.float32) + b3_ref[...]
    raw_ref[...] = raw

    @pl.when(pl.program_id(0) == 0)
    def _():
        acc_ref[...] = jnp.zeros_like(acc_ref)

    acc_ref[...] += jnp.broadcast_to(jnp.sum(raw), acc_ref.shape)


def _tc_mono_head(n_mono, w1, b1, w2, b2, w3, b3, blk):
    N = n_mono.shape[0]
    return pl.pallas_call(
        _mono_head_body,
        grid=(N // blk,),
        in_specs=[
            pl.BlockSpec((blk, 64), lambda i: (i, 0)),
            pl.BlockSpec((64, 64), lambda i: (0, 0)),
            pl.BlockSpec((1, 64), lambda i: (0, 0)),
            pl.BlockSpec((64, 64), lambda i: (0, 0)),
            pl.BlockSpec((1, 64), lambda i: (0, 0)),
            pl.BlockSpec((64, 1), lambda i: (0, 0)),
            pl.BlockSpec((1, 1), lambda i: (0, 0)),
        ],
        out_specs=[
            pl.BlockSpec((blk, 1), lambda i: (i, 0)),
            pl.BlockSpec((8, 128), lambda i: (0, 0)),
        ],
        out_shape=[
            jax.ShapeDtypeStruct((N, 1), jnp.float32),
            jax.ShapeDtypeStruct((8, 128), jnp.float32),
        ],
    )(n_mono, w1, b1, w2, b2, w3, b3)


def _mean_sub_body(raw_ref, acc_ref, o_ref, *, count):
    total = jnp.sum(acc_ref[...]) / (8.0 * 128.0)
    o_ref[...] = raw_ref[...] - total / count


def _tc_mean_sub(raw, acc, blk):
    N = raw.shape[0]
    return pl.pallas_call(
        functools.partial(_mean_sub_body, count=float(N)),
        grid=(N // blk,),
        in_specs=[
            pl.BlockSpec((blk, 1), lambda i: (i, 0)),
            pl.BlockSpec((8, 128), lambda i: (0, 0)),
        ],
        out_specs=pl.BlockSpec((blk, 1), lambda i: (i, 0)),
        out_shape=jax.ShapeDtypeStruct((N, 1), jnp.float32),
    )(raw, acc)


def _edge_head_body(hs_ref, hr_ref, ed_ref,
                    w1d_ref, b1d_ref, w2d_ref, b2d_ref, w3d_ref, b3d_ref,
                    w1q_ref, b1q_ref, w2q_ref, b2q_ref, w3q_ref, b3q_ref,
                    o_ref):
    ed = ed_ref[...]

    def head(off, w1, b1, w2, b2, w3, b3):
        x = jnp.concatenate(
            [hs_ref[:, off:off + 64], hr_ref[:, off:off + 64], ed], axis=1
        )
        h = _mila(jnp.dot(x, w1[...], preferred_element_type=jnp.float32) + b1[...])
        h = _mila(jnp.dot(h, w2[...], preferred_element_type=jnp.float32) + b2[...])
        return jnp.dot(h, w3[...], preferred_element_type=jnp.float32) + b3[...]

    wd = head(0, w1d_ref, b1d_ref, w2d_ref, b2d_ref, w3d_ref, b3d_ref)
    wq = head(64, w1q_ref, b1q_ref, w2q_ref, b2q_ref, w3q_ref, b3q_ref)

    v = hs_ref[:, 128:131] - hr_ref[:, 128:131]
    vx, vy, vz = v[:, 0:1], v[:, 1:2], v[:, 2:3]
    tr3 = (vx * vx + vy * vy + vz * vz) * (1.0 / 3.0)
    q00 = wq * (vx * vx - tr3)
    q11 = wq * (vy * vy - tr3)
    q22 = wq * (vz * vz - tr3)
    q01 = wq * (vx * vy)
    q02 = wq * (vx * vz)
    q12 = wq * (vy * vz)
    zero = jnp.zeros_like(wd)
    o_ref[...] = jnp.concatenate(
        [wd * vx, wd * vy, wd * vz,
         q00, q01, q02, q01, q11, q12, q02, q12, q22,
         zero, zero, zero, zero],
        axis=1,
    )


def _tc_edge_head(hs, hr, edges, pd, pq, blk):
    E = hs.shape[0]
    w1d, b1d, w2d, b2d, w3d, b3d = pd
    w1q, b1q, w2q, b2q, w3q, b3q = pq
    wspec = lambda shp: pl.BlockSpec(shp, lambda i: (0, 0))
    return pl.pallas_call(
        _edge_head_body,
        grid=(E // blk,),
        in_specs=[
            pl.BlockSpec((blk, 144), lambda i: (i, 0)),
            pl.BlockSpec((blk, 144), lambda i: (i, 0)),
            pl.BlockSpec((blk, 32), lambda i: (i, 0)),
            wspec((160, 64)), wspec((1, 64)), wspec((64, 64)), wspec((1, 64)),
            wspec((64, 1)), wspec((1, 1)),
            wspec((160, 64)), wspec((1, 64)), wspec((64, 64)), wspec((1, 64)),
            wspec((64, 1)), wspec((1, 1)),
        ],
        out_specs=pl.BlockSpec((blk, 16), lambda i: (i, 0)),
        out_shape=jax.ShapeDtypeStruct((E, 16), jnp.float32),
    )(hs, hr, edges, w1d, b1d, w2d, b2d, w3d, b3d,
      w1q, b1q, w2q, b2q, w3q, b3q)


_BRANCHES = ("mono", "dipo", "quad")


def _stack_step(params, t, which, li):
    w = jnp.stack([params["gn"][br][t][which][li][0] for br in _BRANCHES])
    b = jnp.stack([params["gn"][br][t][which][li][1][None, :] for br in _BRANCHES])
    return w, b


def _head_params(params, br):
    out = []
    for (w, b) in params["out"][br]:
        out.append(w)
        out.append(b[None, :])
    return tuple(out)


def kernel(nodes, edges, coordinates, params, senders, receivers):
    N = nodes.shape[0]
    E = edges.shape[0]

    senders = senders.astype(jnp.int32)
    receivers = receivers.astype(jnp.int32)
    senders2 = senders.reshape(E // _K, _K)
    receivers2 = receivers.reshape(E // _K, _K)

    wn = jnp.concatenate(
        [params["emb"][br]["node"][0][0] for br in _BRANCHES], axis=1
    )
    wn = jnp.pad(wn, ((0, 1), (0, 0)))
    bn = jnp.concatenate(
        [params["emb"][br]["node"][0][1] for br in _BRANCHES]
    )[None, :]
    we = jnp.concatenate(
        [params["emb"][br]["edge"][0][0] for br in _BRANCHES], axis=1
    )
    be = jnp.concatenate(
        [params["emb"][br]["edge"][0][1] for br in _BRANCHES]
    )[None, :]

    nodes8 = jnp.pad(nodes, ((0, 0), (0, 1)))
    n3 = _tc_embed(nodes8, wn, bn, 2000)

    EH = E // 2
    CH = EH // _K
    s2a, s2b = senders2[:CH], senders2[CH:]
    r2a, r2b = receivers2[:CH], receivers2[CH:]
    eA = _tc_embed(edges[:EH], we, be, 2000)
    eB = _tc_embed(edges[EH:], we, be, 2000)

    gatherH = _make_sc_gather2(N, 192, EH, jnp.float32, 1)
    scatterH = _make_sc_scatter(N, 192, EH, 32, 1)
    zeros32 = jnp.zeros((N, 32), jnp.float32)

    for t in range(4):
        w1e, b1e = _stack_step(params, t, "edge", 0)
        w2e, b2e = _stack_step(params, t, "edge", 1)
        w1n, b1n = _stack_step(params, t, "node", 0)
        w2n, b2n = _stack_step(params, t, "node", 1)

        gsA, grA = gatherH(n3, s2a, r2a)
        n3b, _ = lax.optimization_barrier((n3, gsA))
        gsB, grB = gatherH(n3b, s2b, r2b)
        eA = _tc_edge_step(eA, grA, gsA, w1e, b1e, w2e, b2e, 2000)
        eB = _tc_edge_step(eB, grB, gsB, w1e, b1e, w2e, b2e, 2000)
        eAx, _ = lax.optimization_barrier((eA, gsB))
        aggA = scatterH(eAx, r2a, zeros32)
        eBx, _ = lax.optimization_barrier((eB, aggA))
        aggB = scatterH(eBx, r2b, zeros32)
        n3 = _tc_node_step(aggA, aggB, n3, w1n, b1n, w2n, b2n, 2000)

    raw, acc = _tc_mono_head(n3[:, 0:64], *_head_params(params, "mono"), 2000)
    monopoles = _tc_mean_sub(raw, acc, 2000)

    H = jnp.concatenate(
        [n3[:, 64:192], coordinates, jnp.zeros((N, 13), jnp.float32)], axis=1
    )
    gather144 = _make_sc_gather2(N, 144, E, jnp.float32, 1)
    hs, hr = gather144(H, senders2, receivers2)
    ew = _tc_edge_head(hs, hr, edges,
                       _head_params(params, "dipo"),
                       _head_params(params, "quad"), 2000)
    scatter16 = _make_sc_scatter(N, 16, E, 8, 5)
    zeros8 = jnp.zeros((N, 8), jnp.float32)
    agg16 = scatter16(ew, receivers2, zeros8)

    dipoles = agg16[:, 0:3]
    quadrupoles = agg16[:, 3:12].reshape(N, 3, 3)
    return (monopoles, dipoles, quadrupoles)
